# Initial kernel scaffold; baseline (speedup 1.0000x reference)
import jax
import jax.numpy as jnp
from jax.experimental import pallas as pl


def kernel(means3d, opacities, covariances, features):
    raise NotImplementedError("write your pallas kernel here")


# speedup vs baseline: 1.0850x; 1.0850x over previous
import jax
import jax.numpy as jnp
from jax.experimental import pallas as pl
from jax.experimental.pallas import tpu as pltpu

_VMIN = (-50.0, -50.0, -5.0)
_VMAX = (50.0, 50.0, 3.0)
_VOX = 0.5
_GRID = (200, 200, 16)
_K = 8
_OPACITY_THRESH = 1e-4
_EPS = 1e-6

_B = 16
_NT = 8
_TX = 25
_XPAD = 33
_L = 512
_YW = 16


def _vox_kernel(x0s, yas, fms,
                means, opac, cov9, feat, idxf, span,
                oref, w_s):
    t = pl.program_id(0)
    c = pl.program_id(1)

    @pl.when(c == 0)
    def _init():
        oref[...] = jnp.zeros_like(oref)

    f32 = jnp.float32
    B = means.shape[0]

    def col(ref, j):
        return ref[:, j].reshape(B, 1, 1, 1)

    a = col(cov9, 0); b = col(cov9, 1); cc = col(cov9, 2)
    e = col(cov9, 4); f = col(cov9, 5); i = col(cov9, 8)
    det = a * (e * i - f * f) - b * (b * i - cc * f) + cc * (b * f - cc * e)
    rdet = 1.0 / det
    ixx = (e * i - f * f) * rdet
    iyy = (a * i - cc * cc) * rdet
    izz = (a * e - b * b) * rdet
    ixy = (cc * f - b * i) * rdet
    ixz = (b * f - cc * e) * rdet
    iyz = (b * cc - a * f) * rdet

    dx = jax.lax.broadcasted_iota(jnp.int32, (1, _K, 1, 1), 1).astype(f32)
    dyw = jax.lax.broadcasted_iota(jnp.int32, (1, 1, _YW, 1), 2).astype(f32)
    lane = jax.lax.broadcasted_iota(jnp.int32, (1, 1, 1, _L), 3)
    dz = jnp.where(lane < 384, lane // 16, lane - 384).astype(f32)

    mx = col(means, 0); my = col(means, 1); mz = col(means, 2)
    x0f = col(idxf, 0); y0f = col(idxf, 1); z0f = col(idxf, 2)
    sx = col(span, 0); sy = col(span, 1); sz = col(span, 2)
    op = opac[:, 0].reshape(B, 1, 1, 1)

    yaf = jnp.floor(y0f * 0.125) * 8.0
    ddx = (x0f + dx) * _VOX + (_VMIN[0] + 0.5 * _VOX) - mx
    ddy = (yaf + dyw) * _VOX + (_VMIN[1] + 0.5 * _VOX) - my
    ddz = (dz + 0.5) * _VOX + _VMIN[2] - mz

    vy = yaf + dyw - y0f
    vz = dz - z0f
    valid = ((dx <= sx).astype(f32)
             * ((vy >= 0.0) & (vy <= sy)).astype(f32)
             * ((vz >= 0.0) & (vz <= sz)).astype(f32))
    maha = (ixx * ddx * ddx + iyy * ddy * ddy + izz * ddz * ddz
            + 2.0 * (ixy * ddx * ddy + ixz * ddx * ddz + iyz * ddy * ddz))
    w = op * jnp.exp(-0.5 * maha) * valid

    featl = jnp.concatenate(
        [jnp.tile(feat[...], (1, 24)),
         jnp.ones((B, 24), f32),
         jnp.zeros((B, _L - 408), f32)], axis=1).reshape(B, 1, 1, _L)
    w_s[...] = w * featl

    def body(g, _):
        gi = c * B + g
        x0 = x0s[gi]
        ya8 = yas[gi] * 8
        fm = fms[gi]
        lx = x0 - t * _TX
        in_tile = (x0 >= t * _TX) & ((x0 < (t + 1) * _TX) | (t == _NT - 1))

        @pl.when((fm == 1) & in_tile)
        def _scatter():
            di = (0, pl.ds(lx, _K), pl.ds(ya8, _YW), slice(None))
            oref[di] = oref[di] + w_s[g]

        return 0

    jax.lax.fori_loop(0, B, body, 0)


def kernel(means3d, opacities, covariances, features):
    N = means3d.shape[0]
    C = features.shape[-1]
    f32 = jnp.float32
    vmin = jnp.asarray(_VMIN, f32)
    vmax = jnp.asarray(_VMAX, f32)
    gshape = jnp.asarray(_GRID, jnp.int32)

    sigma = jnp.sqrt(jnp.stack(
        [covariances[:, 0, 0], covariances[:, 1, 1], covariances[:, 2, 2]], -1))
    bmin = means3d - 3.0 * sigma
    bmax = means3d + 3.0 * sigma
    fmask = (jnp.all(bmax > vmin, axis=1) & jnp.all(bmin < vmax, axis=1)
             & (opacities[:, 0] > _OPACITY_THRESH))

    bmin_c = jnp.clip(bmin, vmin, vmax)
    bmax_c = jnp.clip(bmax, vmin, vmax)
    idx_min = jnp.maximum(((bmin_c - vmin) / _VOX).astype(jnp.int32), 0)
    idx_max = jnp.minimum(((bmax_c - vmin) / _VOX).astype(jnp.int32), gshape - 1)
    span = (idx_max - idx_min).astype(f32)

    x0s = idx_min[:, 0]
    yas = idx_min[:, 1] // 8
    fms = fmask.astype(jnp.int32)
    idxf = idx_min.astype(f32)
    cov9 = covariances.reshape(N, 9)

    nc = N // _B
    grid_spec = pltpu.PrefetchScalarGridSpec(
        num_scalar_prefetch=3,
        grid=(_NT, nc),
        in_specs=[
            pl.BlockSpec((_B, 3), lambda t, c, *_: (c, 0)),
            pl.BlockSpec((_B, 1), lambda t, c, *_: (c, 0)),
            pl.BlockSpec((_B, 9), lambda t, c, *_: (c, 0)),
            pl.BlockSpec((_B, C), lambda t, c, *_: (c, 0)),
            pl.BlockSpec((_B, 3), lambda t, c, *_: (c, 0)),
            pl.BlockSpec((_B, 3), lambda t, c, *_: (c, 0)),
        ],
        out_specs=[
            pl.BlockSpec((1, _XPAD, 208, _L), lambda t, c, *_: (t, 0, 0, 0)),
        ],
        scratch_shapes=[
            pltpu.VMEM((_B, _K, _YW, _L), f32),
        ],
    )
    acc = pl.pallas_call(
        _vox_kernel,
        grid_spec=grid_spec,
        out_shape=[
            jax.ShapeDtypeStruct((_NT, _XPAD, 208, _L), f32),
        ],
        compiler_params=pltpu.CompilerParams(
            dimension_semantics=("parallel", "arbitrary")),
    )(x0s, yas, fms,
      means3d, opacities, cov9, features, idxf, span)[0]

    full = jnp.zeros((256, 208, _L), f32)
    for t in range(_NT):
        full = full.at[t * _TX:t * _TX + _XPAD].add(acc[t])

    density = full[:200, :200, 384:400][..., None]
    feats = full[:200, :200, :256].reshape(200, 200, 16, C)
    grid_feats = feats / jnp.maximum(density, _EPS)
    return (density, grid_feats)
